# baseline (speedup 1.0000x reference)
import jax
import jax.numpy as jnp
from jax.experimental import pallas as pl
from jax.experimental.pallas import tpu as pltpu


def _ddpm_body(ts_ref, sab_ref, somab_ref, x_ref, n_ref, o_ref):
    i = pl.program_id(0)
    t = ts_ref[i]
    a = sab_ref[t]
    b = somab_ref[t]
    o_ref[...] = a * x_ref[...] + b * n_ref[...]


def kernel(x0, noise, time_steps, sqrt_alpha_bars, sqrt_one_minus_alpha_bars):
    B = x0.shape[0]
    feat = x0.size // B
    R = feat // 128
    x = x0.reshape(B, R, 128)
    n = noise.reshape(B, R, 128)
    ts = time_steps.astype(jnp.int32)
    out = pl.pallas_call(
        _ddpm_body,
        grid=(B,),
        in_specs=[
            pl.BlockSpec(memory_space=pltpu.SMEM),
            pl.BlockSpec(memory_space=pltpu.SMEM),
            pl.BlockSpec(memory_space=pltpu.SMEM),
            pl.BlockSpec((1, R, 128), lambda i: (i, 0, 0)),
            pl.BlockSpec((1, R, 128), lambda i: (i, 0, 0)),
        ],
        out_specs=pl.BlockSpec((1, R, 128), lambda i: (i, 0, 0)),
        out_shape=jax.ShapeDtypeStruct((B, R, 128), x0.dtype),
    )(ts, sqrt_alpha_bars, sqrt_one_minus_alpha_bars, x, n)
    return out.reshape(x0.shape)


# speedup vs baseline: 3.3099x; 3.3099x over previous
import jax
import jax.numpy as jnp
from jax.experimental import pallas as pl
from jax.experimental.pallas import tpu as pltpu


def _ddpm_body(ts_ref, sab_ref, somab_ref, x_ref, n_ref, o_ref):
    i = pl.program_id(0)
    t = ts_ref[i]
    a = sab_ref[t]
    b = somab_ref[t]
    o_ref[...] = a * x_ref[...] + b * n_ref[...]


def kernel(x0, noise, time_steps, sqrt_alpha_bars, sqrt_one_minus_alpha_bars):
    B, C, H, W = x0.shape
    ts = time_steps.astype(jnp.int32)
    out = pl.pallas_call(
        _ddpm_body,
        grid=(B,),
        in_specs=[
            pl.BlockSpec(memory_space=pltpu.SMEM),
            pl.BlockSpec(memory_space=pltpu.SMEM),
            pl.BlockSpec(memory_space=pltpu.SMEM),
            pl.BlockSpec((1, C, H, W), lambda i: (i, 0, 0, 0)),
            pl.BlockSpec((1, C, H, W), lambda i: (i, 0, 0, 0)),
        ],
        out_specs=pl.BlockSpec((1, C, H, W), lambda i: (i, 0, 0, 0)),
        out_shape=jax.ShapeDtypeStruct((B, C, H, W), x0.dtype),
    )(ts, sqrt_alpha_bars, sqrt_one_minus_alpha_bars, x0, noise)
    return out


# speedup vs baseline: 4.7071x; 1.4221x over previous
import jax
import jax.numpy as jnp
from jax.experimental import pallas as pl
from jax.experimental.pallas import tpu as pltpu


_SAMPLES_PER_STEP = 4


def _ddpm_body(ts_ref, sab_ref, somab_ref, x_ref, n_ref, o_ref):
    i = pl.program_id(0)
    for j in range(_SAMPLES_PER_STEP):
        t = ts_ref[i * _SAMPLES_PER_STEP + j]
        a = sab_ref[t]
        b = somab_ref[t]
        o_ref[j] = a * x_ref[j] + b * n_ref[j]


def kernel(x0, noise, time_steps, sqrt_alpha_bars, sqrt_one_minus_alpha_bars):
    B, C, H, W = x0.shape
    ts = time_steps.astype(jnp.int32)
    out = pl.pallas_call(
        _ddpm_body,
        grid=(B // _SAMPLES_PER_STEP,),
        in_specs=[
            pl.BlockSpec(memory_space=pltpu.SMEM),
            pl.BlockSpec(memory_space=pltpu.SMEM),
            pl.BlockSpec(memory_space=pltpu.SMEM),
            pl.BlockSpec((_SAMPLES_PER_STEP, C, H, W), lambda i: (i, 0, 0, 0)),
            pl.BlockSpec((_SAMPLES_PER_STEP, C, H, W), lambda i: (i, 0, 0, 0)),
        ],
        out_specs=pl.BlockSpec((_SAMPLES_PER_STEP, C, H, W), lambda i: (i, 0, 0, 0)),
        out_shape=jax.ShapeDtypeStruct((B, C, H, W), x0.dtype),
    )(ts, sqrt_alpha_bars, sqrt_one_minus_alpha_bars, x0, noise)
    return out
